# baseline (speedup 1.0000x reference)
import functools

import numpy as np
import jax
import jax.numpy as jnp
from jax import lax
from jax.experimental import pallas as pl
from jax.experimental.pallas import tpu as pltpu
from jax.experimental.pallas import tpu_sc as plsc

N = 10000
E = 320000
D = 128
ED = 16
H = 256
L = 3

NC = 2
NS = 16
HC = H // NC
HW = H // 2
HWH = HC // 2
EPC = E // NC
EPT1 = E // (NC * NS)
EPT = E // NS
CH = 80
NCH1 = EPT1 // CH
NCH = EPT // CH
NPAD = 10240
NPT = NPAD // NS
NZ = 128

_pos = np.arange(HC)
_CH_OF_POS = (32 * (_pos // 32) + 2 * (_pos % 16) + ((_pos % 32) // 16)).astype(np.int32)
_POS_OF_CH = np.argsort(_CH_OF_POS).astype(np.int32)
_pos2 = np.arange(H)
_CH_OF_POS_H = (32 * (_pos2 // 32) + 2 * (_pos2 % 16) + ((_pos2 % 32) // 16)).astype(np.int32)
_POS_OF_CH_H = np.argsort(_CH_OF_POS_H).astype(np.int32)

_mesh = plsc.VectorSubcoreMesh(core_axis_name="c", subcore_axis_name="s")


def _unpack2(u):
    maskhi = jnp.int32(-65536)
    lo = lax.bitcast_convert_type(lax.shift_left(u, 16), jnp.float32)
    hi = lax.bitcast_convert_type(lax.bitwise_and(u, maskhi), jnp.float32)
    return lo, hi


def _pack2(lo, hi):
    maskhi = jnp.int32(-65536)
    rbias = jnp.int32(0x8000)
    ul = lax.bitcast_convert_type(lo, jnp.int32)
    uh = lax.bitcast_convert_type(hi, jnp.int32)
    rl = lax.shift_right_logical(ul + rbias, 16)
    rh = lax.bitwise_and(uh + rbias, maskhi)
    return lax.bitwise_or(rl, rh)


@functools.partial(
    pl.kernel,
    mesh=_mesh,
    out_type=(
        jax.ShapeDtypeStruct((2, E // 2, HW), jnp.int32),
        jax.ShapeDtypeStruct((NC, NS, 2, H), jnp.float32),
    ),
    scratch_types=(
        pltpu.VMEM((CH,), jnp.int32),
        pltpu.VMEM((CH,), jnp.int32),
        pltpu.VMEM((CH, HW), jnp.int32),
        pltpu.VMEM((CH, HW), jnp.int32),
        pltpu.VMEM((CH, HW), jnp.int32),
        pltpu.VMEM((CH // 2, HW), jnp.int32),
        pltpu.VMEM((CH // 2, HW), jnp.int32),
        pltpu.VMEM((2, H), jnp.float32),
        pltpu.SemaphoreType.DMA,
        pltpu.SemaphoreType.DMA,
    ),
)
def _p1(dst1, src1, a_t, b_t, c_t, mpre, stats, di, si, ab, bb, cb, m0, m1, st, sem_a, sem_b):
    cid = lax.axis_index("c")
    sid = lax.axis_index("s")
    zero = jnp.zeros((16,), jnp.float32)

    def chunk(k, accs):
        base = cid * EPC + sid * EPT1 + k * CH
        pltpu.sync_copy(dst1.at[pl.ds(base, CH)], di)
        pltpu.sync_copy(src1.at[pl.ds(base, CH)], si)
        ga = pltpu.async_copy(a_t.at[di], ab, sem_a)
        gb = pltpu.async_copy(b_t.at[si], bb, sem_b)
        pltpu.sync_copy(c_t.at[pl.ds(base, CH)], cb)
        ga.wait()
        gb.wait()

        def rowpair(p, rc):
            vs = list(rc)
            for half in range(2):
                r = 2 * p + half
                for j in range(HW // 16):
                    sl = pl.ds(j * 16, 16)
                    al, ah = _unpack2(ab[r, sl])
                    bl, bh = _unpack2(bb[r, sl])
                    cl, chh = _unpack2(cb[r, sl])
                    ve = al + bl + cl
                    vo = ah + bh + chh
                    w = _pack2(ve, vo)
                    if j < HW // 32:
                        m0[p, pl.ds(half * HWH + j * 16, 16)] = w
                    else:
                        m1[p, pl.ds(half * HWH + (j - HW // 32) * 16, 16)] = w
                    vs[2 * j] = vs[2 * j] + ve
                    vs[2 * j + 1] = vs[2 * j + 1] + vo
                    vs[16 + 2 * j] = vs[16 + 2 * j] + ve * ve
                    vs[16 + 2 * j + 1] = vs[16 + 2 * j + 1] + vo * vo
            return tuple(vs)

        accs = lax.fori_loop(0, CH // 2, rowpair, accs)
        base2 = cid * (EPC // 2) + sid * (EPT1 // 2) + k * (CH // 2)
        pltpu.sync_copy(m0, mpre.at[0, pl.ds(base2, CH // 2)])
        pltpu.sync_copy(m1, mpre.at[1, pl.ds(base2, CH // 2)])
        return accs

    accs = lax.fori_loop(0, NCH1, chunk, tuple(zero for _ in range(32)))
    for q in range(16):
        st[0, pl.ds(q * 16, 16)] = accs[q]
        st[1, pl.ds(q * 16, 16)] = accs[16 + q]
    pltpu.sync_copy(st, stats.at[cid, sid])


@functools.partial(
    pl.kernel,
    mesh=_mesh,
    out_type=jax.ShapeDtypeStruct((NC, NPAD, HC), jnp.float32),
    scratch_types=(
        pltpu.VMEM((CH,), jnp.int32),
        pltpu.VMEM((CH // 2, HW), jnp.int32),
        pltpu.VMEM((CH, HC), jnp.float32),
        pltpu.VMEM((2, HC), jnp.float32),
        pltpu.VMEM((NZ, HC), jnp.float32),
        pltpu.VMEM_SHARED((NPAD, HC), jnp.float32),
        pltpu.SemaphoreType.DMA,
    ),
)
def _p2(dst1, mpre, ss, s_out, di, vb, vb32, ssb, zb, s_sh, sem):
    cid = lax.axis_index("c")
    sid = lax.axis_index("s")
    zero = jnp.zeros((16,), jnp.float32)

    def zrow(r, _):
        for j in range(HC // 16):
            zb[r, pl.ds(j * 16, 16)] = zero
        return 0

    lax.fori_loop(0, NZ, zrow, 0)
    nbase = sid * NPT
    for z in range(NPT // NZ):
        pltpu.sync_copy(zb, s_sh.at[pl.ds(nbase + z * NZ, NZ)])
    plsc.subcore_barrier()

    pltpu.sync_copy(ss.at[cid], ssb)
    sc = [ssb[0, pl.ds(q * 16, 16)] for q in range(HC // 16)]
    sh = [ssb[1, pl.ds(q * 16, 16)] for q in range(HC // 16)]

    def chunk(k, _):
        base = sid * EPT + k * CH
        pltpu.sync_copy(dst1.at[pl.ds(base, CH)], di)
        base2 = sid * (EPT // 2) + k * (CH // 2)
        pltpu.sync_copy(mpre.at[cid, pl.ds(base2, CH // 2)], vb)

        def rowpair(p, _2):
            for half in range(2):
                r = 2 * p + half
                for j in range(HWH // 16):
                    lo, hi = _unpack2(vb[p, pl.ds(half * HWH + j * 16, 16)])
                    q0, q1 = 2 * j, 2 * j + 1
                    vb32[r, pl.ds(q0 * 16, 16)] = jnp.maximum(lo * sc[q0] + sh[q0], 0.0)
                    vb32[r, pl.ds(q1 * 16, 16)] = jnp.maximum(hi * sc[q1] + sh[q1], 0.0)
            return 0

        lax.fori_loop(0, CH // 2, rowpair, 0)
        pltpu.sync_copy(vb32, s_sh.at[di], add=True)
        return 0

    lax.fori_loop(0, NCH, chunk, 0)
    plsc.subcore_barrier()
    for z in range(NPT // NZ):
        pltpu.sync_copy(s_sh.at[pl.ds(nbase + z * NZ, NZ)], zb)
        pltpu.sync_copy(zb, s_out.at[cid, pl.ds(nbase + z * NZ, NZ)])


@functools.partial(
    pl.kernel,
    mesh=_mesh,
    out_type=jax.ShapeDtypeStruct((NC, NPAD, HC), jnp.float32),
    scratch_types=(
        pltpu.VMEM((CH,), jnp.int32),
        pltpu.VMEM((CH, HC), jnp.float32),
        pltpu.VMEM((NZ, HC), jnp.float32),
        pltpu.VMEM_SHARED((NPAD, HC), jnp.float32),
    ),
)
def _cnt(dst1, c_out, di, ob, zb, c_sh):
    cid = lax.axis_index("c")
    sid = lax.axis_index("s")
    zero = jnp.zeros((16,), jnp.float32)
    one = jnp.full((16,), 1.0, jnp.float32)

    def fillz(r, _):
        for j in range(HC // 16):
            zb[r, pl.ds(j * 16, 16)] = zero
        return 0

    lax.fori_loop(0, NZ, fillz, 0)
    nbase = sid * NPT
    for z in range(NPT // NZ):
        pltpu.sync_copy(zb, c_sh.at[pl.ds(nbase + z * NZ, NZ)])
    plsc.subcore_barrier()

    def fillo(r, _):
        for j in range(HC // 16):
            ob[r, pl.ds(j * 16, 16)] = one
        return 0

    lax.fori_loop(0, CH, fillo, 0)

    def chunk(k, _):
        base = sid * EPT + k * CH
        pltpu.sync_copy(dst1.at[pl.ds(base, CH)], di)
        pltpu.sync_copy(ob, c_sh.at[di], add=True)
        return 0

    lax.fori_loop(0, NCH, chunk, 0)
    plsc.subcore_barrier()
    for z in range(NPT // NZ):
        pltpu.sync_copy(c_sh.at[pl.ds(nbase + z * NZ, NZ)], zb)
        pltpu.sync_copy(zb, c_out.at[cid, pl.ds(nbase + z * NZ, NZ)])


def _pack_i32(x):
    xb = x.astype(jnp.bfloat16)
    return lax.bitcast_convert_type(xb.reshape(*x.shape[:-1], x.shape[-1] // 2, 2), jnp.int32)


def kernel(node_features, edge_index, edge_attr, msg_w1, msg_b1, msg_bn_g,
           msg_bn_b, msg_w2, msg_b2, upd_w, upd_b, ln_g, ln_b):
    src = edge_index[0]
    dst = edge_index[1]

    cnt = _cnt(dst)[0, :N, 0]
    inv = 1.0 / jnp.maximum(cnt, 1.0)
    has = (cnt > 0.0).astype(jnp.float32)

    h = node_features
    for l in range(L):
        A = h @ msg_w1[l][:D]
        B = h @ msg_w1[l][D:2 * D]
        C = edge_attr @ msg_w1[l][2 * D:] + msg_b1[l]

        mpre, stats_p = _p1(dst, src, _pack_i32(A), _pack_i32(B), _pack_i32(C))
        stats = stats_p.sum(axis=(0, 1))
        s1 = stats[0][_POS_OF_CH_H]
        s2 = stats[1][_POS_OF_CH_H]
        mu = s1 / E
        var = s2 / E - mu * mu
        scale = msg_bn_g[l] * lax.rsqrt(var + 1e-5)
        shift = msg_bn_b[l] - mu * scale
        ss = jnp.stack([jnp.stack([scale[:HC][_CH_OF_POS], shift[:HC][_CH_OF_POS]]),
                        jnp.stack([scale[HC:][_CH_OF_POS], shift[HC:][_CH_OF_POS]])])

        S2 = _p2(dst, mpre, ss)
        Sfull = jnp.concatenate([S2[0, :N][:, _POS_OF_CH], S2[1, :N][:, _POS_OF_CH]], axis=1)
        aggm = Sfull * inv[:, None]
        agg = aggm @ msg_w2[l] + msg_b2[l] * has[:, None]

        u = h @ upd_w[l][:D] + agg @ upd_w[l][D:] + upd_b[l]
        h = h + u
        mu2 = h.mean(axis=-1, keepdims=True)
        var2 = h.var(axis=-1, keepdims=True)
        h = (h - mu2) * lax.rsqrt(var2 + 1e-5) * ln_g[l] + ln_b[l]
    return h


# speedup vs baseline: 2.0014x; 2.0014x over previous
import functools

import jax
import jax.numpy as jnp
from jax import lax
from jax.experimental import pallas as pl
from jax.experimental.pallas import tpu as pltpu
from jax.experimental.pallas import tpu_sc as plsc

N = 10000
E = 320000
D = 128
ED = 16
H = 256
L = 3

NC = 2
NS = 16
HC = H // NC
EPT = E // NS
CH = 200
NCH = EPT // CH
NPAD = 10240
NPT = NPAD // NS
NZ = 128

BN = 400
BE = 2000

_mesh = plsc.VectorSubcoreMesh(core_axis_name="c", subcore_axis_name="s")


@functools.partial(
    pl.kernel,
    mesh=_mesh,
    out_type=(
        jax.ShapeDtypeStruct((NC, E, HC), jnp.float32),
        jax.ShapeDtypeStruct((NC, NS, 2, HC), jnp.float32),
    ),
    scratch_types=(
        pltpu.VMEM((CH,), jnp.int32),
        pltpu.VMEM((CH,), jnp.int32),
        pltpu.VMEM((CH, HC), jnp.float32),
        pltpu.VMEM((CH, HC), jnp.float32),
        pltpu.VMEM((CH, HC), jnp.float32),
        pltpu.VMEM((2, HC), jnp.float32),
        pltpu.SemaphoreType.DMA,
        pltpu.SemaphoreType.DMA,
    ),
)
def _p1(dst2, src2, a_t, b_t, c_t, mpre, stats, di, si, ab, bb, cb, st, sem_a, sem_b):
    cid = lax.axis_index("c")
    sid = lax.axis_index("s")
    zero = jnp.zeros((16,), jnp.float32)

    def chunk(k, accs):
        base = sid * EPT + k * CH
        ibase = cid * E + base
        pltpu.sync_copy(dst2.at[pl.ds(ibase, CH)], di)
        pltpu.sync_copy(src2.at[pl.ds(ibase, CH)], si)
        ga = pltpu.async_copy(a_t.at[di], ab, sem_a)
        gb = pltpu.async_copy(b_t.at[si], bb, sem_b)
        pltpu.sync_copy(c_t.at[cid, pl.ds(base, CH)], cb)
        ga.wait()
        gb.wait()

        def row(r, rc):
            vs = list(rc)
            for j in range(HC // 16):
                sl = pl.ds(j * 16, 16)
                v = ab[r, sl] + bb[r, sl] + cb[r, sl]
                cb[r, sl] = v
                vs[j] = vs[j] + v
                vs[j + 8] = vs[j + 8] + v * v
            return tuple(vs)

        accs = lax.fori_loop(0, CH, row, accs)
        pltpu.sync_copy(cb, mpre.at[cid, pl.ds(base, CH)])
        return accs

    accs = lax.fori_loop(0, NCH, chunk, tuple(zero for _ in range(16)))
    for j in range(HC // 16):
        st[0, pl.ds(j * 16, 16)] = accs[j]
        st[1, pl.ds(j * 16, 16)] = accs[j + 8]
    pltpu.sync_copy(st, stats.at[cid, sid])


@functools.partial(
    pl.kernel,
    mesh=_mesh,
    out_type=jax.ShapeDtypeStruct((NC, NPAD, HC), jnp.float32),
    scratch_types=(
        pltpu.VMEM((CH,), jnp.int32),
        pltpu.VMEM((CH, HC), jnp.float32),
        pltpu.VMEM((2, HC), jnp.float32),
        pltpu.VMEM((NZ, HC), jnp.float32),
        pltpu.VMEM_SHARED((NPAD, HC), jnp.float32),
        pltpu.SemaphoreType.DMA,
    ),
)
def _p2(dst1, mpre, ss, s_out, di, vb, ssb, zb, s_sh, sem):
    cid = lax.axis_index("c")
    sid = lax.axis_index("s")
    zero = jnp.zeros((16,), jnp.float32)

    def zrow(r, _):
        for j in range(HC // 16):
            zb[r, pl.ds(j * 16, 16)] = zero
        return 0

    lax.fori_loop(0, NZ, zrow, 0)
    nbase = sid * NPT
    for z in range(NPT // NZ):
        pltpu.sync_copy(zb, s_sh.at[pl.ds(nbase + z * NZ, NZ)])
    plsc.subcore_barrier()

    pltpu.sync_copy(ss.at[cid], ssb)
    sc = [ssb[0, pl.ds(j * 16, 16)] for j in range(HC // 16)]
    sh = [ssb[1, pl.ds(j * 16, 16)] for j in range(HC // 16)]

    def chunk(k, _):
        base = sid * EPT + k * CH
        pltpu.sync_copy(dst1.at[pl.ds(base, CH)], di)
        pltpu.sync_copy(mpre.at[cid, pl.ds(base, CH)], vb)

        def row(r, _2):
            for j in range(HC // 16):
                sl = pl.ds(j * 16, 16)
                vb[r, sl] = jnp.maximum(vb[r, sl] * sc[j] + sh[j], 0.0)
            return 0

        lax.fori_loop(0, CH, row, 0)
        pltpu.sync_copy(vb, s_sh.at[di], add=True)
        return 0

    lax.fori_loop(0, NCH, chunk, 0)
    plsc.subcore_barrier()
    for z in range(NPT // NZ):
        pltpu.sync_copy(s_sh.at[pl.ds(nbase + z * NZ, NZ)], zb)
        pltpu.sync_copy(zb, s_out.at[cid, pl.ds(nbase + z * NZ, NZ)])


@functools.partial(
    pl.kernel,
    mesh=_mesh,
    out_type=jax.ShapeDtypeStruct((NC, NPAD, HC), jnp.float32),
    scratch_types=(
        pltpu.VMEM((CH,), jnp.int32),
        pltpu.VMEM((CH, HC), jnp.float32),
        pltpu.VMEM((NZ, HC), jnp.float32),
        pltpu.VMEM_SHARED((NPAD, HC), jnp.float32),
    ),
)
def _cnt(dst1, c_out, di, ob, zb, c_sh):
    cid = lax.axis_index("c")
    sid = lax.axis_index("s")
    zero = jnp.zeros((16,), jnp.float32)
    one = jnp.full((16,), 1.0, jnp.float32)

    def fillz(r, _):
        for j in range(HC // 16):
            zb[r, pl.ds(j * 16, 16)] = zero
        return 0

    lax.fori_loop(0, NZ, fillz, 0)
    nbase = sid * NPT
    for z in range(NPT // NZ):
        pltpu.sync_copy(zb, c_sh.at[pl.ds(nbase + z * NZ, NZ)])
    plsc.subcore_barrier()

    def fillo(r, _):
        for j in range(HC // 16):
            ob[r, pl.ds(j * 16, 16)] = one
        return 0

    lax.fori_loop(0, CH, fillo, 0)

    def chunk(k, _):
        base = sid * EPT + k * CH
        pltpu.sync_copy(dst1.at[pl.ds(base, CH)], di)
        pltpu.sync_copy(ob, c_sh.at[di], add=True)
        return 0

    lax.fori_loop(0, NCH, chunk, 0)
    plsc.subcore_barrier()
    for z in range(NPT // NZ):
        pltpu.sync_copy(c_sh.at[pl.ds(nbase + z * NZ, NZ)], zb)
        pltpu.sync_copy(zb, c_out.at[cid, pl.ds(nbase + z * NZ, NZ)])


def _pre_k(h_ref, w_ref, a_ref, b_ref):
    p = lax.dot_general(h_ref[...], w_ref[...], (((1,), (0,)), ((), ())),
                        preferred_element_type=jnp.float32)
    a_ref[0] = p[:, 0:HC]
    a_ref[1] = p[:, HC:2 * HC]
    b_ref[0] = p[:, 2 * HC:3 * HC]
    b_ref[1] = p[:, 3 * HC:4 * HC]


def _pre_call(h, wab):
    return pl.pallas_call(
        _pre_k,
        grid=(N // BN,),
        in_specs=[
            pl.BlockSpec((BN, D), lambda i: (i, 0)),
            pl.BlockSpec((D, 2 * H), lambda i: (0, 0)),
        ],
        out_specs=[
            pl.BlockSpec((2, BN, HC), lambda i: (0, i, 0)),
            pl.BlockSpec((2, BN, HC), lambda i: (0, i, 0)),
        ],
        out_shape=[
            jax.ShapeDtypeStruct((2, N, HC), jnp.float32),
            jax.ShapeDtypeStruct((2, N, HC), jnp.float32),
        ],
    )(h, wab)


def _c_k(ea_ref, w_ref, b_ref, c_ref):
    c = lax.dot_general(ea_ref[...], w_ref[...], (((1,), (0,)), ((), ())),
                        preferred_element_type=jnp.float32) + b_ref[0:1, :]
    c_ref[0] = c[:, 0:HC]
    c_ref[1] = c[:, HC:H]


def _c_call(ea, w1c, b1pad):
    return pl.pallas_call(
        _c_k,
        grid=(E // BE,),
        in_specs=[
            pl.BlockSpec((BE, ED), lambda i: (i, 0)),
            pl.BlockSpec((ED, H), lambda i: (0, 0)),
            pl.BlockSpec((8, H), lambda i: (0, 0)),
        ],
        out_specs=pl.BlockSpec((2, BE, HC), lambda i: (0, i, 0)),
        out_shape=jax.ShapeDtypeStruct((2, E, HC), jnp.float32),
    )(ea, w1c, b1pad)


def _post_k(h_ref, s_ref, c_ref, wuh_ref, w2u_ref, aux_ref, o_ref):
    c0 = c_ref[:, 0:1]
    inv = 1.0 / jnp.maximum(c0, 1.0)
    has = (c0 > 0.0).astype(jnp.float32)
    a0 = s_ref[0] * inv
    a1 = s_ref[1] * inv
    u = (lax.dot_general(h_ref[...], wuh_ref[...], (((1,), (0,)), ((), ())),
                         preferred_element_type=jnp.float32)
         + lax.dot_general(a0, w2u_ref[0], (((1,), (0,)), ((), ())),
                           preferred_element_type=jnp.float32)
         + lax.dot_general(a1, w2u_ref[1], (((1,), (0,)), ((), ())),
                           preferred_element_type=jnp.float32)
         + aux_ref[0:1, :] + aux_ref[1:2, :] * has)
    hn = h_ref[...] + u
    mu = jnp.mean(hn, axis=-1, keepdims=True)
    d = hn - mu
    var = jnp.mean(d * d, axis=-1, keepdims=True)
    o_ref[...] = d * lax.rsqrt(var + 1e-5) * aux_ref[2:3, :] + aux_ref[3:4, :]


def _post_call(h, s2, cnt0, wuh, w2u, aux):
    return pl.pallas_call(
        _post_k,
        grid=(N // BN,),
        in_specs=[
            pl.BlockSpec((BN, D), lambda i: (i, 0)),
            pl.BlockSpec((2, BN, HC), lambda i: (0, i, 0)),
            pl.BlockSpec((BN, HC), lambda i: (i, 0)),
            pl.BlockSpec((D, D), lambda i: (0, 0)),
            pl.BlockSpec((2, HC, D), lambda i: (0, 0, 0)),
            pl.BlockSpec((8, D), lambda i: (0, 0)),
        ],
        out_specs=pl.BlockSpec((BN, D), lambda i: (i, 0)),
        out_shape=jax.ShapeDtypeStruct((N, D), jnp.float32),
    )(h, s2, cnt0, wuh, w2u, aux)


def kernel(node_features, edge_index, edge_attr, msg_w1, msg_b1, msg_bn_g,
           msg_bn_b, msg_w2, msg_b2, upd_w, upd_b, ln_g, ln_b):
    src = edge_index[0]
    dst = edge_index[1]
    dst2 = jnp.concatenate([dst, dst + N])
    src2 = jnp.concatenate([src, src + N])

    cnt0 = _cnt(dst)[0]
    zrow = jnp.zeros((1, D), jnp.float32)

    h = node_features
    for l in range(L):
        wab = jnp.concatenate([msg_w1[l][:D], msg_w1[l][D:2 * D]], axis=1)
        a3, b3 = _pre_call(h, wab)
        b1pad = jnp.concatenate([msg_b1[l][None], jnp.zeros((7, H), jnp.float32)], axis=0)
        c2 = _c_call(edge_attr, msg_w1[l][2 * D:], b1pad)

        mpre, stats_p = _p1(dst2, src2, a3.reshape(2 * N, HC), b3.reshape(2 * N, HC), c2)
        stats = stats_p.sum(axis=1)
        s1 = jnp.concatenate([stats[0, 0], stats[1, 0]])
        s2m = jnp.concatenate([stats[0, 1], stats[1, 1]])
        mu = s1 / E
        var = s2m / E - mu * mu
        scale = msg_bn_g[l] * lax.rsqrt(var + 1e-5)
        shift = msg_bn_b[l] - mu * scale
        ss = jnp.stack([jnp.stack([scale[:HC], shift[:HC]]),
                        jnp.stack([scale[HC:], shift[HC:]])])

        s2 = _p2(dst, mpre, ss)

        w2u = msg_w2[l] @ upd_w[l][D:]
        w2u_split = jnp.stack([w2u[:HC], w2u[HC:]])
        aux = jnp.concatenate([upd_b[l][None], (msg_b2[l] @ upd_w[l][D:])[None],
                               ln_g[l][None], ln_b[l][None],
                               zrow, zrow, zrow, zrow], axis=0)
        h = _post_call(h, s2, cnt0, upd_w[l][:D], w2u_split, aux)
    return h


# speedup vs baseline: 2.5774x; 1.2878x over previous
import functools

import jax
import jax.numpy as jnp
from jax import lax
from jax.experimental import pallas as pl
from jax.experimental.pallas import tpu as pltpu
from jax.experimental.pallas import tpu_sc as plsc

N = 10000
E = 320000
D = 128
ED = 16
H = 256
L = 3

NC = 2
NS = 16
HC = H // NC
EPT = E // NS
CH = 200
NCH = EPT // CH
CH1 = 160
NCH1 = EPT // CH1
CH2 = 200
NCH2 = EPT // CH2
NPAD = 10240
NPT = NPAD // NS
NZ = 128

BN = 400
BE = 2000

_mesh = plsc.VectorSubcoreMesh(core_axis_name="c", subcore_axis_name="s")


@functools.partial(
    pl.kernel,
    mesh=_mesh,
    out_type=(
        jax.ShapeDtypeStruct((NC, E, HC), jnp.float32),
        jax.ShapeDtypeStruct((NC, NS, 2, HC), jnp.float32),
    ),
    scratch_types=(
        pltpu.VMEM((CH1,), jnp.int32),
        pltpu.VMEM((CH1,), jnp.int32),
        pltpu.VMEM((CH1, HC), jnp.float32),
        pltpu.VMEM((CH1, HC), jnp.float32),
        pltpu.VMEM((CH1, HC), jnp.float32),
        pltpu.VMEM((CH1,), jnp.int32),
        pltpu.VMEM((CH1,), jnp.int32),
        pltpu.VMEM((CH1, HC), jnp.float32),
        pltpu.VMEM((CH1, HC), jnp.float32),
        pltpu.VMEM((CH1, HC), jnp.float32),
        pltpu.VMEM((2, HC), jnp.float32),
        pltpu.SemaphoreType.DMA,
        pltpu.SemaphoreType.DMA,
        pltpu.SemaphoreType.DMA,
        pltpu.SemaphoreType.DMA,
        pltpu.SemaphoreType.DMA,
        pltpu.SemaphoreType.DMA,
    ),
)
def _p1(dst2, src2, a_t, b_t, c_t, mpre, stats,
        di0, si0, ab0, bb0, cb0, di1, si1, ab1, bb1, cb1, st,
        sa0, sb0, sc0, sa1, sb1, sc1):
    cid = lax.axis_index("c")
    sid = lax.axis_index("s")
    zero = jnp.zeros((16,), jnp.float32)
    sets = ((di0, si0, ab0, bb0, cb0, sa0, sb0, sc0),
            (di1, si1, ab1, bb1, cb1, sa1, sb1, sc1))

    def issue(k, s):
        di, si, ab, bb, cb, sa, sb, scm = s
        base = sid * EPT + k * CH1
        ibase = cid * E + base
        pltpu.sync_copy(dst2.at[pl.ds(ibase, CH1)], di)
        pltpu.sync_copy(src2.at[pl.ds(ibase, CH1)], si)
        pltpu.async_copy(a_t.at[di], ab, sa)
        pltpu.async_copy(b_t.at[si], bb, sb)
        pltpu.async_copy(c_t.at[cid, pl.ds(base, CH1)], cb, scm)

    def drain(k, s):
        di, si, ab, bb, cb, sa, sb, scm = s
        base = sid * EPT + k * CH1
        pltpu.make_async_copy(a_t.at[di], ab, sa).wait()
        pltpu.make_async_copy(b_t.at[si], bb, sb).wait()
        pltpu.make_async_copy(c_t.at[cid, pl.ds(base, CH1)], cb, scm).wait()

    def compute(k, s, accs):
        di, si, ab, bb, cb, sa, sb, scm = s

        def row(r, rc):
            vs = list(rc)
            for j in range(HC // 16):
                sl = pl.ds(j * 16, 16)
                v = ab[r, sl] + bb[r, sl] + cb[r, sl]
                cb[r, sl] = v
                vs[j] = vs[j] + v
                vs[j + 8] = vs[j + 8] + v * v
            return tuple(vs)

        accs = lax.fori_loop(0, CH1, row, accs)
        base = sid * EPT + k * CH1
        pltpu.sync_copy(cb, mpre.at[cid, pl.ds(base, CH1)])
        return accs

    issue(0, sets[0])

    def pair(k2, accs):
        k0 = 2 * k2
        issue(k0 + 1, sets[1])
        drain(k0, sets[0])
        accs = compute(k0, sets[0], accs)
        issue(k0 + 2, sets[0])
        drain(k0 + 1, sets[1])
        accs = compute(k0 + 1, sets[1], accs)
        return accs

    accs = lax.fori_loop(0, (NCH1 - 1) // 2, pair, tuple(zero for _ in range(16)))
    last = NCH1 - 1
    drain(last, sets[0])
    accs = compute(last, sets[0], accs)
    for j in range(HC // 16):
        st[0, pl.ds(j * 16, 16)] = accs[j]
        st[1, pl.ds(j * 16, 16)] = accs[j + 8]
    pltpu.sync_copy(st, stats.at[cid, sid])


@functools.partial(
    pl.kernel,
    mesh=_mesh,
    out_type=jax.ShapeDtypeStruct((NC, NPAD, HC), jnp.float32),
    scratch_types=(
        pltpu.VMEM((CH2,), jnp.int32),
        pltpu.VMEM((CH2, HC), jnp.float32),
        pltpu.VMEM((2, HC), jnp.float32),
        pltpu.VMEM((NZ, HC), jnp.float32),
        pltpu.VMEM_SHARED((NPAD, HC), jnp.float32),
        pltpu.SemaphoreType.DMA,
    ),
)
def _p2(dst1, mpre, ss, s_out, di, vb, ssb, zb, s_sh, sem):
    cid = lax.axis_index("c")
    sid = lax.axis_index("s")
    zero = jnp.zeros((16,), jnp.float32)

    def zrow(r, _):
        for j in range(HC // 16):
            zb[r, pl.ds(j * 16, 16)] = zero
        return 0

    lax.fori_loop(0, NZ, zrow, 0)
    nbase = sid * NPT
    for z in range(NPT // NZ):
        pltpu.sync_copy(zb, s_sh.at[pl.ds(nbase + z * NZ, NZ)])
    plsc.subcore_barrier()

    pltpu.sync_copy(ss.at[cid], ssb)
    sc = [ssb[0, pl.ds(j * 16, 16)] for j in range(HC // 16)]
    sh = [ssb[1, pl.ds(j * 16, 16)] for j in range(HC // 16)]

    def chunk(k, _):
        base = sid * EPT + k * CH2
        pltpu.sync_copy(dst1.at[pl.ds(base, CH2)], di)
        pltpu.sync_copy(mpre.at[cid, pl.ds(base, CH2)], vb)

        def row(r, _2):
            for j in range(HC // 16):
                sl = pl.ds(j * 16, 16)
                vb[r, sl] = jnp.maximum(vb[r, sl] * sc[j] + sh[j], 0.0)
            return 0

        lax.fori_loop(0, CH2, row, 0)
        pltpu.sync_copy(vb, s_sh.at[di], add=True)
        return 0

    lax.fori_loop(0, NCH2, chunk, 0)
    plsc.subcore_barrier()
    for z in range(NPT // NZ):
        pltpu.sync_copy(s_sh.at[pl.ds(nbase + z * NZ, NZ)], zb)
        pltpu.sync_copy(zb, s_out.at[cid, pl.ds(nbase + z * NZ, NZ)])


@functools.partial(
    pl.kernel,
    mesh=_mesh,
    out_type=jax.ShapeDtypeStruct((NC, NPAD, HC), jnp.float32),
    scratch_types=(
        pltpu.VMEM((CH,), jnp.int32),
        pltpu.VMEM((CH, HC), jnp.float32),
        pltpu.VMEM((NZ, HC), jnp.float32),
        pltpu.VMEM_SHARED((NPAD, HC), jnp.float32),
    ),
)
def _cnt(dst1, c_out, di, ob, zb, c_sh):
    cid = lax.axis_index("c")
    sid = lax.axis_index("s")
    zero = jnp.zeros((16,), jnp.float32)
    one = jnp.full((16,), 1.0, jnp.float32)

    def fillz(r, _):
        for j in range(HC // 16):
            zb[r, pl.ds(j * 16, 16)] = zero
        return 0

    lax.fori_loop(0, NZ, fillz, 0)
    nbase = sid * NPT
    for z in range(NPT // NZ):
        pltpu.sync_copy(zb, c_sh.at[pl.ds(nbase + z * NZ, NZ)])
    plsc.subcore_barrier()

    def fillo(r, _):
        for j in range(HC // 16):
            ob[r, pl.ds(j * 16, 16)] = one
        return 0

    lax.fori_loop(0, CH, fillo, 0)

    def chunk(k, _):
        base = sid * EPT + k * CH
        pltpu.sync_copy(dst1.at[pl.ds(base, CH)], di)
        pltpu.sync_copy(ob, c_sh.at[di], add=True)
        return 0

    lax.fori_loop(0, NCH, chunk, 0)
    plsc.subcore_barrier()
    for z in range(NPT // NZ):
        pltpu.sync_copy(c_sh.at[pl.ds(nbase + z * NZ, NZ)], zb)
        pltpu.sync_copy(zb, c_out.at[cid, pl.ds(nbase + z * NZ, NZ)])


def _pre_k(h_ref, w_ref, a_ref, b_ref):
    p = lax.dot_general(h_ref[...], w_ref[...], (((1,), (0,)), ((), ())),
                        preferred_element_type=jnp.float32)
    a_ref[0] = p[:, 0:HC]
    a_ref[1] = p[:, HC:2 * HC]
    b_ref[0] = p[:, 2 * HC:3 * HC]
    b_ref[1] = p[:, 3 * HC:4 * HC]


def _pre_call(h, wab):
    return pl.pallas_call(
        _pre_k,
        grid=(N // BN,),
        in_specs=[
            pl.BlockSpec((BN, D), lambda i: (i, 0)),
            pl.BlockSpec((D, 2 * H), lambda i: (0, 0)),
        ],
        out_specs=[
            pl.BlockSpec((2, BN, HC), lambda i: (0, i, 0)),
            pl.BlockSpec((2, BN, HC), lambda i: (0, i, 0)),
        ],
        out_shape=[
            jax.ShapeDtypeStruct((2, N, HC), jnp.float32),
            jax.ShapeDtypeStruct((2, N, HC), jnp.float32),
        ],
    )(h, wab)


def _c_k(ea_ref, w_ref, b_ref, c_ref):
    c = lax.dot_general(ea_ref[...], w_ref[...], (((1,), (0,)), ((), ())),
                        preferred_element_type=jnp.float32) + b_ref[0:1, :]
    c_ref[0] = c[:, 0:HC]
    c_ref[1] = c[:, HC:H]


def _c_call(ea, w1c, b1pad):
    return pl.pallas_call(
        _c_k,
        grid=(E // BE,),
        in_specs=[
            pl.BlockSpec((BE, ED), lambda i: (i, 0)),
            pl.BlockSpec((ED, H), lambda i: (0, 0)),
            pl.BlockSpec((8, H), lambda i: (0, 0)),
        ],
        out_specs=pl.BlockSpec((2, BE, HC), lambda i: (0, i, 0)),
        out_shape=jax.ShapeDtypeStruct((2, E, HC), jnp.float32),
    )(ea, w1c, b1pad)


def _post_k(h_ref, s_ref, c_ref, wuh_ref, w2u_ref, aux_ref, o_ref):
    c0 = c_ref[:, 0:1]
    inv = 1.0 / jnp.maximum(c0, 1.0)
    has = (c0 > 0.0).astype(jnp.float32)
    a0 = s_ref[0] * inv
    a1 = s_ref[1] * inv
    u = (lax.dot_general(h_ref[...], wuh_ref[...], (((1,), (0,)), ((), ())),
                         preferred_element_type=jnp.float32)
         + lax.dot_general(a0, w2u_ref[0], (((1,), (0,)), ((), ())),
                           preferred_element_type=jnp.float32)
         + lax.dot_general(a1, w2u_ref[1], (((1,), (0,)), ((), ())),
                           preferred_element_type=jnp.float32)
         + aux_ref[0:1, :] + aux_ref[1:2, :] * has)
    hn = h_ref[...] + u
    mu = jnp.mean(hn, axis=-1, keepdims=True)
    d = hn - mu
    var = jnp.mean(d * d, axis=-1, keepdims=True)
    o_ref[...] = d * lax.rsqrt(var + 1e-5) * aux_ref[2:3, :] + aux_ref[3:4, :]


def _post_call(h, s2, cnt0, wuh, w2u, aux):
    return pl.pallas_call(
        _post_k,
        grid=(N // BN,),
        in_specs=[
            pl.BlockSpec((BN, D), lambda i: (i, 0)),
            pl.BlockSpec((2, BN, HC), lambda i: (0, i, 0)),
            pl.BlockSpec((BN, HC), lambda i: (i, 0)),
            pl.BlockSpec((D, D), lambda i: (0, 0)),
            pl.BlockSpec((2, HC, D), lambda i: (0, 0, 0)),
            pl.BlockSpec((8, D), lambda i: (0, 0)),
        ],
        out_specs=pl.BlockSpec((BN, D), lambda i: (i, 0)),
        out_shape=jax.ShapeDtypeStruct((N, D), jnp.float32),
    )(h, s2, cnt0, wuh, w2u, aux)


def kernel(node_features, edge_index, edge_attr, msg_w1, msg_b1, msg_bn_g,
           msg_bn_b, msg_w2, msg_b2, upd_w, upd_b, ln_g, ln_b):
    src = edge_index[0]
    dst = edge_index[1]
    dst2 = jnp.concatenate([dst, dst + N])
    src2 = jnp.concatenate([src, src + N])

    cnt0 = _cnt(dst)[0]
    zrow = jnp.zeros((1, D), jnp.float32)

    h = node_features
    for l in range(L):
        wab = jnp.concatenate([msg_w1[l][:D], msg_w1[l][D:2 * D]], axis=1)
        a3, b3 = _pre_call(h, wab)
        b1pad = jnp.concatenate([msg_b1[l][None], jnp.zeros((7, H), jnp.float32)], axis=0)
        c2 = _c_call(edge_attr, msg_w1[l][2 * D:], b1pad)

        mpre, stats_p = _p1(dst2, src2, a3.reshape(2 * N, HC), b3.reshape(2 * N, HC), c2)
        stats = stats_p.sum(axis=1)
        s1 = jnp.concatenate([stats[0, 0], stats[1, 0]])
        s2m = jnp.concatenate([stats[0, 1], stats[1, 1]])
        mu = s1 / E
        var = s2m / E - mu * mu
        scale = msg_bn_g[l] * lax.rsqrt(var + 1e-5)
        shift = msg_bn_b[l] - mu * scale
        ss = jnp.stack([jnp.stack([scale[:HC], shift[:HC]]),
                        jnp.stack([scale[HC:], shift[HC:]])])

        s2 = _p2(dst, mpre, ss)

        w2u = msg_w2[l] @ upd_w[l][D:]
        w2u_split = jnp.stack([w2u[:HC], w2u[HC:]])
        aux = jnp.concatenate([upd_b[l][None], (msg_b2[l] @ upd_w[l][D:])[None],
                               ln_g[l][None], ln_b[l][None],
                               zrow, zrow, zrow, zrow], axis=0)
        h = _post_call(h, s2, cnt0, upd_w[l][:D], w2u_split, aux)
    return h


# speedup vs baseline: 2.7711x; 1.0751x over previous
import functools

import jax
import jax.numpy as jnp
from jax import lax
from jax.experimental import pallas as pl
from jax.experimental.pallas import tpu as pltpu
from jax.experimental.pallas import tpu_sc as plsc

N = 10000
E = 320000
D = 128
ED = 16
H = 256
L = 3

NC = 2
NS = 16
HC = H // NC
EPT = E // NS
CH = 200
NCH = EPT // CH
CH1 = 160
NCH1 = EPT // CH1
CH2 = 80
NCH2 = EPT // CH2
NPAD = 10240
NPT = NPAD // NS
NZ = 128

BN = 400
BE = 2000

_mesh = plsc.VectorSubcoreMesh(core_axis_name="c", subcore_axis_name="s")


@functools.partial(
    pl.kernel,
    mesh=_mesh,
    out_type=(
        jax.ShapeDtypeStruct((NC, E, HC), jnp.float32),
        jax.ShapeDtypeStruct((NC, NS, 2, HC), jnp.float32),
    ),
    scratch_types=(
        pltpu.VMEM((CH1,), jnp.int32),
        pltpu.VMEM((CH1,), jnp.int32),
        pltpu.VMEM((CH1, HC), jnp.float32),
        pltpu.VMEM((CH1, HC), jnp.float32),
        pltpu.VMEM((CH1, HC), jnp.float32),
        pltpu.VMEM((CH1,), jnp.int32),
        pltpu.VMEM((CH1,), jnp.int32),
        pltpu.VMEM((CH1, HC), jnp.float32),
        pltpu.VMEM((CH1, HC), jnp.float32),
        pltpu.VMEM((CH1, HC), jnp.float32),
        pltpu.VMEM((2, HC), jnp.float32),
        pltpu.SemaphoreType.DMA,
        pltpu.SemaphoreType.DMA,
        pltpu.SemaphoreType.DMA,
        pltpu.SemaphoreType.DMA,
        pltpu.SemaphoreType.DMA,
        pltpu.SemaphoreType.DMA,
    ),
)
def _p1(dst2, src2, a_t, b_t, c_t, mpre, stats,
        di0, si0, ab0, bb0, cb0, di1, si1, ab1, bb1, cb1, st,
        sa0, sb0, sc0, sa1, sb1, sc1):
    cid = lax.axis_index("c")
    sid = lax.axis_index("s")
    zero = jnp.zeros((16,), jnp.float32)
    sets = ((di0, si0, ab0, bb0, cb0, sa0, sb0, sc0),
            (di1, si1, ab1, bb1, cb1, sa1, sb1, sc1))

    def issue(k, s):
        di, si, ab, bb, cb, sa, sb, scm = s
        base = sid * EPT + k * CH1
        ibase = cid * E + base
        pltpu.sync_copy(dst2.at[pl.ds(ibase, CH1)], di)
        pltpu.sync_copy(src2.at[pl.ds(ibase, CH1)], si)
        pltpu.async_copy(a_t.at[di], ab, sa)
        pltpu.async_copy(b_t.at[si], bb, sb)
        pltpu.async_copy(c_t.at[cid, pl.ds(base, CH1)], cb, scm)

    def drain(k, s):
        di, si, ab, bb, cb, sa, sb, scm = s
        base = sid * EPT + k * CH1
        pltpu.make_async_copy(a_t.at[di], ab, sa).wait()
        pltpu.make_async_copy(b_t.at[si], bb, sb).wait()
        pltpu.make_async_copy(c_t.at[cid, pl.ds(base, CH1)], cb, scm).wait()

    def compute(k, s, accs):
        di, si, ab, bb, cb, sa, sb, scm = s

        def row(r, rc):
            vs = list(rc)
            for j in range(HC // 16):
                sl = pl.ds(j * 16, 16)
                v = ab[r, sl] + bb[r, sl] + cb[r, sl]
                cb[r, sl] = v
                vs[j] = vs[j] + v
                vs[j + 8] = vs[j + 8] + v * v
            return tuple(vs)

        accs = lax.fori_loop(0, CH1, row, accs)
        base = sid * EPT + k * CH1
        pltpu.sync_copy(cb, mpre.at[cid, pl.ds(base, CH1)])
        return accs

    issue(0, sets[0])

    def pair(k2, accs):
        k0 = 2 * k2
        issue(k0 + 1, sets[1])
        drain(k0, sets[0])
        accs = compute(k0, sets[0], accs)
        issue(k0 + 2, sets[0])
        drain(k0 + 1, sets[1])
        accs = compute(k0 + 1, sets[1], accs)
        return accs

    accs = lax.fori_loop(0, (NCH1 - 1) // 2, pair, tuple(zero for _ in range(16)))
    last = NCH1 - 1
    drain(last, sets[0])
    accs = compute(last, sets[0], accs)
    for j in range(HC // 16):
        st[0, pl.ds(j * 16, 16)] = accs[j]
        st[1, pl.ds(j * 16, 16)] = accs[j + 8]
    pltpu.sync_copy(st, stats.at[cid, sid])


@functools.partial(
    pl.kernel,
    mesh=_mesh,
    out_type=jax.ShapeDtypeStruct((NC, NPAD, HC), jnp.float32),
    scratch_types=(
        pltpu.VMEM((CH2,), jnp.int32),
        pltpu.VMEM((CH2, HC), jnp.float32),
        pltpu.VMEM((CH2,), jnp.int32),
        pltpu.VMEM((CH2, HC), jnp.float32),
        pltpu.VMEM((2, HC), jnp.float32),
        pltpu.VMEM((NZ, HC), jnp.float32),
        pltpu.VMEM_SHARED((NPAD, HC), jnp.float32),
        pltpu.SemaphoreType.DMA,
        pltpu.SemaphoreType.DMA,
    ),
)
def _p2(dst1, mpre, ss, s_out, di0, vb0, di1, vb1, ssb, zb, s_sh, sm0, sm1):
    cid = lax.axis_index("c")
    sid = lax.axis_index("s")
    zero = jnp.zeros((16,), jnp.float32)

    def zrow(r, _):
        for j in range(HC // 16):
            zb[r, pl.ds(j * 16, 16)] = zero
        return 0

    lax.fori_loop(0, NZ, zrow, 0)
    nbase = sid * NPT
    for z in range(NPT // NZ):
        pltpu.sync_copy(zb, s_sh.at[pl.ds(nbase + z * NZ, NZ)])
    plsc.subcore_barrier()

    pltpu.sync_copy(ss.at[cid], ssb)
    sc = [ssb[0, pl.ds(j * 16, 16)] for j in range(HC // 16)]
    sh = [ssb[1, pl.ds(j * 16, 16)] for j in range(HC // 16)]
    sets = ((di0, vb0, sm0), (di1, vb1, sm1))

    def issue(k, s):
        di, vb, sm = s
        base = sid * EPT + k * CH2
        pltpu.sync_copy(dst1.at[pl.ds(base, CH2)], di)
        pltpu.async_copy(mpre.at[cid, pl.ds(base, CH2)], vb, sm)

    def drain(k, s):
        di, vb, sm = s
        base = sid * EPT + k * CH2
        pltpu.make_async_copy(mpre.at[cid, pl.ds(base, CH2)], vb, sm).wait()

    def compute(s):
        di, vb, sm = s

        def row(r, _2):
            for j in range(HC // 16):
                sl = pl.ds(j * 16, 16)
                vb[r, sl] = jnp.maximum(vb[r, sl] * sc[j] + sh[j], 0.0)
            return 0

        lax.fori_loop(0, CH2, row, 0)
        pltpu.sync_copy(vb, s_sh.at[di], add=True)

    issue(0, sets[0])

    def pair(k2, _):
        k0 = 2 * k2
        issue(k0 + 1, sets[1])
        drain(k0, sets[0])
        compute(sets[0])
        @pl.when(k0 + 2 < NCH2)
        def _issue_next():
            issue(k0 + 2, sets[0])
        drain(k0 + 1, sets[1])
        compute(sets[1])
        return 0

    lax.fori_loop(0, NCH2 // 2, pair, 0)
    plsc.subcore_barrier()
    for z in range(NPT // NZ):
        pltpu.sync_copy(s_sh.at[pl.ds(nbase + z * NZ, NZ)], zb)
        pltpu.sync_copy(zb, s_out.at[cid, pl.ds(nbase + z * NZ, NZ)])


@functools.partial(
    pl.kernel,
    mesh=_mesh,
    out_type=jax.ShapeDtypeStruct((NC, NPAD, HC), jnp.float32),
    scratch_types=(
        pltpu.VMEM((CH,), jnp.int32),
        pltpu.VMEM((CH, HC), jnp.float32),
        pltpu.VMEM((NZ, HC), jnp.float32),
        pltpu.VMEM_SHARED((NPAD, HC), jnp.float32),
    ),
)
def _cnt(dst1, c_out, di, ob, zb, c_sh):
    cid = lax.axis_index("c")
    sid = lax.axis_index("s")
    zero = jnp.zeros((16,), jnp.float32)
    one = jnp.full((16,), 1.0, jnp.float32)

    def fillz(r, _):
        for j in range(HC // 16):
            zb[r, pl.ds(j * 16, 16)] = zero
        return 0

    lax.fori_loop(0, NZ, fillz, 0)
    nbase = sid * NPT
    for z in range(NPT // NZ):
        pltpu.sync_copy(zb, c_sh.at[pl.ds(nbase + z * NZ, NZ)])
    plsc.subcore_barrier()

    def fillo(r, _):
        for j in range(HC // 16):
            ob[r, pl.ds(j * 16, 16)] = one
        return 0

    lax.fori_loop(0, CH, fillo, 0)

    def chunk(k, _):
        base = sid * EPT + k * CH
        pltpu.sync_copy(dst1.at[pl.ds(base, CH)], di)
        pltpu.sync_copy(ob, c_sh.at[di], add=True)
        return 0

    lax.fori_loop(0, NCH, chunk, 0)
    plsc.subcore_barrier()
    for z in range(NPT // NZ):
        pltpu.sync_copy(c_sh.at[pl.ds(nbase + z * NZ, NZ)], zb)
        pltpu.sync_copy(zb, c_out.at[cid, pl.ds(nbase + z * NZ, NZ)])


def _pre_k(h_ref, w_ref, a_ref, b_ref):
    p = lax.dot_general(h_ref[...], w_ref[...], (((1,), (0,)), ((), ())),
                        preferred_element_type=jnp.float32)
    a_ref[0] = p[:, 0:HC]
    a_ref[1] = p[:, HC:2 * HC]
    b_ref[0] = p[:, 2 * HC:3 * HC]
    b_ref[1] = p[:, 3 * HC:4 * HC]


def _pre_call(h, wab):
    return pl.pallas_call(
        _pre_k,
        grid=(N // BN,),
        in_specs=[
            pl.BlockSpec((BN, D), lambda i: (i, 0)),
            pl.BlockSpec((D, 2 * H), lambda i: (0, 0)),
        ],
        out_specs=[
            pl.BlockSpec((2, BN, HC), lambda i: (0, i, 0)),
            pl.BlockSpec((2, BN, HC), lambda i: (0, i, 0)),
        ],
        out_shape=[
            jax.ShapeDtypeStruct((2, N, HC), jnp.float32),
            jax.ShapeDtypeStruct((2, N, HC), jnp.float32),
        ],
    )(h, wab)


def _c_k(ea_ref, w_ref, b_ref, c_ref):
    c = lax.dot_general(ea_ref[...], w_ref[...], (((1,), (0,)), ((), ())),
                        preferred_element_type=jnp.float32) + b_ref[0:1, :]
    c_ref[0] = c[:, 0:HC]
    c_ref[1] = c[:, HC:H]


def _c_call(ea, w1c, b1pad):
    return pl.pallas_call(
        _c_k,
        grid=(E // BE,),
        in_specs=[
            pl.BlockSpec((BE, ED), lambda i: (i, 0)),
            pl.BlockSpec((ED, H), lambda i: (0, 0)),
            pl.BlockSpec((8, H), lambda i: (0, 0)),
        ],
        out_specs=pl.BlockSpec((2, BE, HC), lambda i: (0, i, 0)),
        out_shape=jax.ShapeDtypeStruct((2, E, HC), jnp.float32),
    )(ea, w1c, b1pad)


def _post_k(h_ref, s_ref, c_ref, wuh_ref, w2u_ref, aux_ref, o_ref):
    c0 = c_ref[:, 0:1]
    inv = 1.0 / jnp.maximum(c0, 1.0)
    has = (c0 > 0.0).astype(jnp.float32)
    a0 = s_ref[0] * inv
    a1 = s_ref[1] * inv
    u = (lax.dot_general(h_ref[...], wuh_ref[...], (((1,), (0,)), ((), ())),
                         preferred_element_type=jnp.float32)
         + lax.dot_general(a0, w2u_ref[0], (((1,), (0,)), ((), ())),
                           preferred_element_type=jnp.float32)
         + lax.dot_general(a1, w2u_ref[1], (((1,), (0,)), ((), ())),
                           preferred_element_type=jnp.float32)
         + aux_ref[0:1, :] + aux_ref[1:2, :] * has)
    hn = h_ref[...] + u
    mu = jnp.mean(hn, axis=-1, keepdims=True)
    d = hn - mu
    var = jnp.mean(d * d, axis=-1, keepdims=True)
    o_ref[...] = d * lax.rsqrt(var + 1e-5) * aux_ref[2:3, :] + aux_ref[3:4, :]


def _post_call(h, s2, cnt0, wuh, w2u, aux):
    return pl.pallas_call(
        _post_k,
        grid=(N // BN,),
        in_specs=[
            pl.BlockSpec((BN, D), lambda i: (i, 0)),
            pl.BlockSpec((2, BN, HC), lambda i: (0, i, 0)),
            pl.BlockSpec((BN, HC), lambda i: (i, 0)),
            pl.BlockSpec((D, D), lambda i: (0, 0)),
            pl.BlockSpec((2, HC, D), lambda i: (0, 0, 0)),
            pl.BlockSpec((8, D), lambda i: (0, 0)),
        ],
        out_specs=pl.BlockSpec((BN, D), lambda i: (i, 0)),
        out_shape=jax.ShapeDtypeStruct((N, D), jnp.float32),
    )(h, s2, cnt0, wuh, w2u, aux)


def kernel(node_features, edge_index, edge_attr, msg_w1, msg_b1, msg_bn_g,
           msg_bn_b, msg_w2, msg_b2, upd_w, upd_b, ln_g, ln_b):
    src = edge_index[0]
    dst = edge_index[1]
    dst2 = jnp.concatenate([dst, dst + N])
    src2 = jnp.concatenate([src, src + N])

    cnt0 = _cnt(dst)[0]
    zrow = jnp.zeros((1, D), jnp.float32)

    h = node_features
    for l in range(L):
        wab = jnp.concatenate([msg_w1[l][:D], msg_w1[l][D:2 * D]], axis=1)
        a3, b3 = _pre_call(h, wab)
        b1pad = jnp.concatenate([msg_b1[l][None], jnp.zeros((7, H), jnp.float32)], axis=0)
        c2 = _c_call(edge_attr, msg_w1[l][2 * D:], b1pad)

        mpre, stats_p = _p1(dst2, src2, a3.reshape(2 * N, HC), b3.reshape(2 * N, HC), c2)
        stats = stats_p.sum(axis=1)
        s1 = jnp.concatenate([stats[0, 0], stats[1, 0]])
        s2m = jnp.concatenate([stats[0, 1], stats[1, 1]])
        mu = s1 / E
        var = s2m / E - mu * mu
        scale = msg_bn_g[l] * lax.rsqrt(var + 1e-5)
        shift = msg_bn_b[l] - mu * scale
        ss = jnp.stack([jnp.stack([scale[:HC], shift[:HC]]),
                        jnp.stack([scale[HC:], shift[HC:]])])

        s2 = _p2(dst, mpre, ss)

        w2u = msg_w2[l] @ upd_w[l][D:]
        w2u_split = jnp.stack([w2u[:HC], w2u[HC:]])
        aux = jnp.concatenate([upd_b[l][None], (msg_b2[l] @ upd_w[l][D:])[None],
                               ln_g[l][None], ln_b[l][None],
                               zrow, zrow, zrow, zrow], axis=0)
        h = _post_call(h, s2, cnt0, upd_w[l][:D], w2u_split, aux)
    return h
